# Initial kernel scaffold; baseline (speedup 1.0000x reference)
import jax
import jax.numpy as jnp
from jax.experimental import pallas as pl


def kernel(user, pos_item, neg_item, user_table, item_table, adj_rows, adj_cols, adj_vals):
    raise NotImplementedError("write your pallas kernel here")


# speedup vs baseline: 5.5165x; 5.5165x over previous
import jax
import jax.numpy as jnp
from jax import lax
from jax.experimental import pallas as pl
from jax.experimental.pallas import tpu as pltpu
from jax.experimental.pallas import tpu_sc as plsc

N_USERS = 50000
N_ITEMS = 50000
N_NODES = 100000
EMB = 32
LAYERS = 3
BATCH = 4096

NC = 2
NS = 16
G = 128
CH_G = 2
GPT = 784
EPAD = GPT * G * NS
HALF = 50000
ACC_ROWS = 50176
ZPT = ACC_ROWS // NS
ROWS_PT = HALF // NS

_mesh = plsc.VectorSubcoreMesh(core_axis_name="c", subcore_axis_name="s")

_GDN = lax.GatherDimensionNumbers(
    offset_dims=(), collapsed_slice_dims=(0,), start_index_map=(0,))


def _bcast_lane(v16, e):
    idx = jnp.full((16, 1), e, jnp.int32)
    return lax.gather(v16, idx, _GDN, (1,),
                      mode=lax.GatherScatterMode.PROMISE_IN_BOUNDS)


def _propagate_body(feat, cols, rows, vals, out, colv, rowv, valv, dstl, buf, acc):
    c = lax.axis_index("c")
    s = lax.axis_index("s")
    off = c * HALF

    zero16 = jnp.zeros((16,), jnp.float32)
    for i in range(G):
        buf[0, i, pl.ds(0, 16)] = zero16
        buf[0, i, pl.ds(16, 16)] = zero16
    zbase = s * ZPT
    for z in range(ZPT // G):
        pltpu.sync_copy(buf.at[0], acc.at[pl.ds(zbase + z * G, G)])
    rem = ZPT - (ZPT // G) * G
    if rem:
        pltpu.sync_copy(buf.at[0, pl.ds(0, rem)],
                        acc.at[pl.ds(zbase + (ZPT // G) * G, rem)])
    plsc.subcore_barrier()

    def chunk(ci, carry):
        be = (s * GPT + ci * CH_G) * G
        pltpu.sync_copy(cols.at[pl.ds(be, CH_G * G)], colv)
        pltpu.sync_copy(rows.at[pl.ds(be, CH_G * G)], rowv)
        pltpu.sync_copy(vals.at[pl.ds(be, CH_G * G)], valv)
        for gi in range(CH_G):
            pltpu.sync_copy(feat.at[colv.at[pl.ds(gi * G, G)]], buf.at[gi])
        for gi in range(CH_G):
            for q in range(8):
                j0 = gi * G + q * 16
                v16 = valv[pl.ds(j0, 16)]
                d16 = rowv[pl.ds(j0, 16)] - off
                okm = (d16 >= 0) & (d16 < HALF)
                dstl[gi][pl.ds(q * 16, 16)] = jnp.where(okm, d16, HALF)
                for e in range(16):
                    v = _bcast_lane(v16, e)
                    r = q * 16 + e
                    buf[gi, r, pl.ds(0, 16)] = buf[gi, r, pl.ds(0, 16)] * v
                    buf[gi, r, pl.ds(16, 16)] = buf[gi, r, pl.ds(16, 16)] * v
        for gi in range(CH_G):
            pltpu.sync_copy(buf.at[gi], acc.at[dstl[gi]], add=True)
        return carry

    lax.fori_loop(0, GPT // CH_G, chunk, 0)
    plsc.subcore_barrier()

    ob = s * ZPT

    @pl.when(s < NS - 1)
    def _copy_full():
        pltpu.sync_copy(acc.at[pl.ds(ob, ZPT)], out.at[pl.ds(off + ob, ZPT)])

    @pl.when(s == NS - 1)
    def _copy_tail():
        pltpu.sync_copy(acc.at[pl.ds((NS - 1) * ZPT, HALF - (NS - 1) * ZPT)],
                        out.at[pl.ds(off + (NS - 1) * ZPT, HALF - (NS - 1) * ZPT)])


_propagate = pl.kernel(
    _propagate_body,
    out_type=jax.ShapeDtypeStruct((N_NODES, EMB), jnp.float32),
    mesh=_mesh,
    compiler_params=pltpu.CompilerParams(use_tc_tiling_on_sc=False),
    scratch_types=[
        pltpu.VMEM((CH_G * G,), jnp.int32),
        pltpu.VMEM((CH_G * G,), jnp.int32),
        pltpu.VMEM((CH_G * G,), jnp.float32),
        [pltpu.VMEM((G,), jnp.int32) for _ in range(CH_G)],
        pltpu.VMEM((CH_G, G, EMB), jnp.float32),
        pltpu.VMEM_SHARED((ACC_ROWS, EMB), jnp.float32),
    ],
)


def _finalize_body(f0, f1, f2, f3, uidx, pidx, nidx, ue, pe, ne,
                   iv, iv2, ba, bb, bc, bd):
    c = lax.axis_index("c")
    s = lax.axis_index("s")
    w = s * NC + c

    for idx_hbm, out_hbm, base_off in ((uidx, ue, 0), (pidx, pe, HALF), (nidx, ne, HALF)):
        pltpu.sync_copy(idx_hbm.at[w], iv)
        for q in range(8):
            j0 = q * 16
            iv2[pl.ds(j0, 16)] = iv[0, pl.ds(j0, 16)] + base_off
        for arr, b in ((f0, ba), (f1, bb), (f2, bc), (f3, bd)):
            pltpu.sync_copy(arr.at[iv2], b)

        def mean_rows(q, carry):
            for jj in range(16):
                for h in range(2):
                    sl = pl.ds(16 * h, 16)
                    m = (ba[q * 16 + jj, sl] + bb[q * 16 + jj, sl]
                         + bc[q * 16 + jj, sl] + bd[q * 16 + jj, sl]) * 0.25
                    ba[q * 16 + jj, sl] = m
            return carry

        lax.fori_loop(0, G // 16, mean_rows, 0)
        pltpu.sync_copy(ba, out_hbm.at[pl.ds(w * G, G)])


_finalize = pl.kernel(
    _finalize_body,
    out_type=(
        jax.ShapeDtypeStruct((BATCH, EMB), jnp.float32),
        jax.ShapeDtypeStruct((BATCH, EMB), jnp.float32),
        jax.ShapeDtypeStruct((BATCH, EMB), jnp.float32),
    ),
    mesh=_mesh,
    compiler_params=pltpu.CompilerParams(use_tc_tiling_on_sc=False),
    scratch_types=[
        pltpu.VMEM((1, G), jnp.int32),
        pltpu.VMEM((G,), jnp.int32),
        pltpu.VMEM((G, EMB), jnp.float32),
        pltpu.VMEM((G, EMB), jnp.float32),
        pltpu.VMEM((G, EMB), jnp.float32),
        pltpu.VMEM((G, EMB), jnp.float32),
    ],
)


def kernel(user, pos_item, neg_item, user_table, item_table,
           adj_rows, adj_cols, adj_vals):
    node0 = jnp.concatenate([user_table, item_table], axis=0)
    e = adj_rows.shape[0]
    pad = EPAD - e
    cols = jnp.pad(adj_cols.astype(jnp.int32), (0, pad))
    rows = jnp.pad(adj_rows.astype(jnp.int32), (0, pad))
    vals = jnp.pad(adj_vals, (0, pad))

    feats = [node0]
    f = node0
    for _ in range(LAYERS):
        f = _propagate(f, cols, rows, vals)
        feats.append(f)

    u2 = user.astype(jnp.int32).reshape(-1, 1, G)
    p2 = pos_item.astype(jnp.int32).reshape(-1, 1, G)
    n2 = neg_item.astype(jnp.int32).reshape(-1, 1, G)
    return _finalize(feats[0], feats[1], feats[2], feats[3], u2, p2, n2)


# speedup vs baseline: 7.8845x; 1.4293x over previous
import jax
import jax.numpy as jnp
from jax import lax
from jax.experimental import pallas as pl
from jax.experimental.pallas import tpu as pltpu
from jax.experimental.pallas import tpu_sc as plsc

N_USERS = 50000
N_ITEMS = 50000
N_NODES = 100000
EMB = 32
LAYERS = 3
BATCH = 4096

NC = 2
NS = 16
G = 128
SB = 4
NBLK = 196
GPT = 784
EPAD = GPT * G * NS
HALF = 50000
ACC_ROWS = 50176
ZPT = ACC_ROWS // NS
ROWS_PT = HALF // NS

_mesh = plsc.VectorSubcoreMesh(core_axis_name="c", subcore_axis_name="s")

_GDN = lax.GatherDimensionNumbers(
    offset_dims=(), collapsed_slice_dims=(0,), start_index_map=(0,))


def _bcast_lane(v16, e):
    idx = jnp.full((16, 1), e, jnp.int32)
    return lax.gather(v16, idx, _GDN, (1,),
                      mode=lax.GatherScatterMode.PROMISE_IN_BOUNDS)


def _propagate_body(feat, cols, rows, vals, out, colv, rowv, valv, dstl, buf, acc,
                    lsem, gsem, ssem):
    c = lax.axis_index("c")
    s = lax.axis_index("s")
    off = c * HALF

    zero16 = jnp.zeros((16,), jnp.float32)
    for i in range(G):
        buf[0, i, pl.ds(0, 16)] = zero16
        buf[0, i, pl.ds(16, 16)] = zero16
    zbase = s * ZPT
    for z in range(ZPT // G):
        pltpu.sync_copy(buf.at[0], acc.at[pl.ds(zbase + z * G, G)])
    rem = ZPT - (ZPT // G) * G
    if rem:
        pltpu.sync_copy(buf.at[0, pl.ds(0, rem)],
                        acc.at[pl.ds(zbase + (ZPT // G) * G, rem)])
    plsc.subcore_barrier()

    def fire_lin(b, slot):
        be = (s * GPT + b * SB) * G
        pltpu.async_copy(cols.at[pl.ds(be, SB * G)], colv.at[slot], lsem)
        pltpu.async_copy(rows.at[pl.ds(be, SB * G)], rowv.at[slot], lsem)
        pltpu.async_copy(vals.at[pl.ds(be, SB * G)], valv.at[slot], lsem)

    def wait_lin(slot):
        pltpu.make_async_copy(cols.at[pl.ds(0, SB * G)], colv.at[slot], lsem).wait()
        pltpu.make_async_copy(rows.at[pl.ds(0, SB * G)], rowv.at[slot], lsem).wait()
        pltpu.make_async_copy(vals.at[pl.ds(0, SB * G)], valv.at[slot], lsem).wait()

    fire_lin(0, 0)

    def block(b, carry):
        slot = lax.rem(b, 2)
        wait_lin(slot)

        @pl.when(b + 1 < NBLK)
        def _next_lin():
            fire_lin(b + 1, 1 - slot)

        for gi in range(SB):
            for q in range(8):
                d16 = rowv[slot, pl.ds(gi * G + q * 16, 16)] - off
                okm = (d16 >= 0) & (d16 < HALF)
                dstl_p = [dstl[gi], dstl[SB + gi]]
                for p in range(2):
                    @pl.when(slot == p)
                    def _wr(p=p, d16=d16, okm=okm, gi=gi, q=q):
                        dstl_p[p][pl.ds(q * 16, 16)] = jnp.where(okm, d16, HALF)

        gdescs = []
        for gi in range(SB):
            @pl.when(b > 0)
            def _drain_prev(gi=gi):
                pltpu.make_async_copy(buf.at[gi], acc.at[dstl[gi]], ssem).wait()
            gdescs.append(pltpu.async_copy(
                feat.at[colv.at[slot, pl.ds(gi * G, G)]], buf.at[gi], gsem))

        for gi in range(SB):
            gdescs[gi].wait()
            for q in range(8):
                v16 = valv[slot, pl.ds(gi * G + q * 16, 16)]
                for e in range(16):
                    v = _bcast_lane(v16, e)
                    r = q * 16 + e
                    buf[gi, r, pl.ds(0, 16)] = buf[gi, r, pl.ds(0, 16)] * v
                    buf[gi, r, pl.ds(16, 16)] = buf[gi, r, pl.ds(16, 16)] * v
            for p in range(2):
                @pl.when(slot == p)
                def _sc(p=p, gi=gi):
                    pltpu.async_copy(buf.at[gi], acc.at[dstl[p * SB + gi]],
                                     ssem, add=True)
        return carry

    lax.fori_loop(0, NBLK, block, 0)
    for gi in range(SB):
        pltpu.make_async_copy(buf.at[gi], acc.at[dstl[gi]], ssem).wait()
    plsc.subcore_barrier()

    ob = s * ZPT

    @pl.when(s < NS - 1)
    def _copy_full():
        pltpu.sync_copy(acc.at[pl.ds(ob, ZPT)], out.at[pl.ds(off + ob, ZPT)])

    @pl.when(s == NS - 1)
    def _copy_tail():
        pltpu.sync_copy(acc.at[pl.ds((NS - 1) * ZPT, HALF - (NS - 1) * ZPT)],
                        out.at[pl.ds(off + (NS - 1) * ZPT, HALF - (NS - 1) * ZPT)])


_propagate = pl.kernel(
    _propagate_body,
    out_type=jax.ShapeDtypeStruct((N_NODES, EMB), jnp.float32),
    mesh=_mesh,
    compiler_params=pltpu.CompilerParams(use_tc_tiling_on_sc=False),
    scratch_types=[
        pltpu.VMEM((2, SB * G), jnp.int32),
        pltpu.VMEM((2, SB * G), jnp.int32),
        pltpu.VMEM((2, SB * G), jnp.float32),
        [pltpu.VMEM((G,), jnp.int32) for _ in range(2 * SB)],
        pltpu.VMEM((SB, G, EMB), jnp.float32),
        pltpu.VMEM_SHARED((ACC_ROWS, EMB), jnp.float32),
        pltpu.SemaphoreType.DMA,
        pltpu.SemaphoreType.DMA,
        pltpu.SemaphoreType.DMA,
    ],
)


def _finalize_body(f0, f1, f2, f3, uidx, pidx, nidx, ue, pe, ne,
                   iv, iv2, ba, bb, bc, bd):
    c = lax.axis_index("c")
    s = lax.axis_index("s")
    w = s * NC + c

    for idx_hbm, out_hbm, base_off in ((uidx, ue, 0), (pidx, pe, HALF), (nidx, ne, HALF)):
        pltpu.sync_copy(idx_hbm.at[w], iv)
        for q in range(8):
            j0 = q * 16
            iv2[pl.ds(j0, 16)] = iv[0, pl.ds(j0, 16)] + base_off
        for arr, b in ((f0, ba), (f1, bb), (f2, bc), (f3, bd)):
            pltpu.sync_copy(arr.at[iv2], b)

        def mean_rows(q, carry):
            for jj in range(16):
                for h in range(2):
                    sl = pl.ds(16 * h, 16)
                    m = (ba[q * 16 + jj, sl] + bb[q * 16 + jj, sl]
                         + bc[q * 16 + jj, sl] + bd[q * 16 + jj, sl]) * 0.25
                    ba[q * 16 + jj, sl] = m
            return carry

        lax.fori_loop(0, G // 16, mean_rows, 0)
        pltpu.sync_copy(ba, out_hbm.at[pl.ds(w * G, G)])


_finalize = pl.kernel(
    _finalize_body,
    out_type=(
        jax.ShapeDtypeStruct((BATCH, EMB), jnp.float32),
        jax.ShapeDtypeStruct((BATCH, EMB), jnp.float32),
        jax.ShapeDtypeStruct((BATCH, EMB), jnp.float32),
    ),
    mesh=_mesh,
    compiler_params=pltpu.CompilerParams(use_tc_tiling_on_sc=False),
    scratch_types=[
        pltpu.VMEM((1, G), jnp.int32),
        pltpu.VMEM((G,), jnp.int32),
        pltpu.VMEM((G, EMB), jnp.float32),
        pltpu.VMEM((G, EMB), jnp.float32),
        pltpu.VMEM((G, EMB), jnp.float32),
        pltpu.VMEM((G, EMB), jnp.float32),
    ],
)


def kernel(user, pos_item, neg_item, user_table, item_table,
           adj_rows, adj_cols, adj_vals):
    node0 = jnp.concatenate([user_table, item_table], axis=0)
    e = adj_rows.shape[0]
    pad = EPAD - e
    cols = jnp.pad(adj_cols.astype(jnp.int32), (0, pad))
    rows = jnp.pad(adj_rows.astype(jnp.int32), (0, pad))
    vals = jnp.pad(adj_vals, (0, pad))

    feats = [node0]
    f = node0
    for _ in range(LAYERS):
        f = _propagate(f, cols, rows, vals)
        feats.append(f)

    u2 = user.astype(jnp.int32).reshape(-1, 1, G)
    p2 = pos_item.astype(jnp.int32).reshape(-1, 1, G)
    n2 = neg_item.astype(jnp.int32).reshape(-1, 1, G)
    return _finalize(feats[0], feats[1], feats[2], feats[3], u2, p2, n2)
